# baseline (speedup 1.0000x reference)
import functools

import jax
import jax.numpy as jnp
from jax import lax
from jax.experimental import pallas as pl
from jax.experimental.pallas import tpu as pltpu
from jax.experimental.pallas import tpu_sc as plsc

_NC = 2
_NS = 16
_NW = _NC * _NS
_L = 16


def _build_transpose(V: int, D: int):
    assert D == 32 and V % 4 == 0
    n_full = V // 128
    main_iters = n_full // _NW
    rem_full = n_full % _NW
    ragged = V % 128
    tail_rows = ragged // 4
    if ragged:
        assert ragged % 4 == 0 and tail_rows % 8 == 0
    mesh = plsc.VectorSubcoreMesh(core_axis_name="c", subcore_axis_name="s")

    def transpose_kernel_body(tT_hbm, tail_hbm, t250_hbm, ibuf, obuf, sem0, sem1):
        wid = lax.axis_index("s") * _NC + lax.axis_index("c")
        sems = (sem0, sem1)
        zi = lax.iota(jnp.int32, 16)
        rvecs = (zi, zi + 16)
        czero = zi * 0

        def shuffle(src, dst, m_rows):
            for m in range(m_rows):
                for h in range(8):
                    cvec = czero + (4 * m + h // 2)
                    v = plsc.load_gather(src, [rvecs[h % 2], cvec])
                    dst[m, pl.ds(16 * h, 16)] = v

        def blk_of(k):
            return wid + k * _NW

        for b in range(2):
            pltpu.async_copy(
                tT_hbm.at[:, pl.ds(blk_of(b) * 128, 128)], ibuf.at[b], sems[b]
            )

        def body(k0, carry):
            for b in range(2):
                k = k0 + b
                c = blk_of(k)
                pltpu.make_async_copy(
                    tT_hbm.at[:, pl.ds(c * 128, 128)], ibuf.at[b], sems[b]
                ).wait()
                shuffle(ibuf.at[b], obuf.at[b], 32)
                pltpu.sync_copy(obuf.at[b], t250_hbm.at[pl.ds(c * 32, 32)])

                @pl.when(k + 2 < main_iters)
                def _():
                    pltpu.async_copy(
                        tT_hbm.at[:, pl.ds(blk_of(k + 2) * 128, 128)],
                        ibuf.at[b],
                        sems[b],
                    )

            return carry

        assert main_iters % 2 == 0
        lax.fori_loop(0, main_iters // 2, lambda i, c: body(i * 2, c), 0)

        if rem_full:
            c_tail = (main_iters * _NW) + wid

            @pl.when(wid < rem_full)
            def _():
                pltpu.sync_copy(
                    tT_hbm.at[:, pl.ds(c_tail * 128, 128)], ibuf.at[0]
                )
                shuffle(ibuf.at[0], obuf.at[0], 32)
                pltpu.sync_copy(
                    obuf.at[0], t250_hbm.at[pl.ds(c_tail * 32, 32)]
                )

        if ragged:

            @pl.when(wid == rem_full)
            def _():
                pltpu.sync_copy(tail_hbm, obuf.at[0, pl.ds(0, tail_rows)])
                pltpu.sync_copy(
                    obuf.at[0, pl.ds(0, tail_rows)],
                    t250_hbm.at[pl.ds(n_full * 32, tail_rows)],
                )

    if ragged:
        body = transpose_kernel_body
    else:
        def body(tT_hbm, t250_hbm, ibuf, obuf, sem0, sem1):
            dummy = None
            return transpose_kernel_body(
                tT_hbm, dummy, t250_hbm, ibuf, obuf, sem0, sem1
            )

    return functools.partial(
        pl.kernel,
        mesh=mesh,
        compiler_params=pltpu.CompilerParams(needs_layout_passes=False),
        out_type=jax.ShapeDtypeStruct((V // 4, 128), jnp.float32),
        scratch_types=[
            pltpu.VMEM((2, 32, 128), jnp.float32),
            pltpu.VMEM((2, 32, 128), jnp.float32),
            pltpu.SemaphoreType.DMA,
            pltpu.SemaphoreType.DMA,
        ],
    )(body)


def _build_gather(V: int, D: int, BATCH: int, FIELDS: int):
    assert D == 32 and BATCH % (128 * _NW) == 0 and FIELDS % 2 == 0
    blocks_per_w = BATCH // (128 * _NW)
    mesh = plsc.VectorSubcoreMesh(core_axis_name="c", subcore_axis_name="s")

    @functools.partial(
        pl.kernel,
        mesh=mesh,
        compiler_params=pltpu.CompilerParams(needs_layout_passes=False),
        out_type=jax.ShapeDtypeStruct((FIELDS, D, BATCH), jnp.float32),
        scratch_types=[
            pltpu.VMEM((FIELDS, 128), jnp.int32),
            pltpu.VMEM((FIELDS, 128), jnp.int32),
            pltpu.VMEM((FIELDS, 128), jnp.int32),
            pltpu.VMEM((2, 128, 128), jnp.float32),
            pltpu.VMEM((2, 32, 128), jnp.float32),
            pltpu.SemaphoreType.DMA,
            pltpu.SemaphoreType.DMA,
        ],
    )
    def gather_kernel(t250_hbm, xT_hbm, out_hbm, idx_v, ridx_v, off_v,
                      gbuf, pbuf, gsem0, gsem1):
        wid = lax.axis_index("s") * _NC + lax.axis_index("c")
        gsems = (gsem0, gsem1)
        zi = lax.iota(jnp.int32, 16)
        erows = tuple(zi + 16 * h for h in range(8))

        def extract(f, b):
            for h in range(8):
                offs = off_v[f, pl.ds(16 * h, 16)]
                for j in range(32):
                    v = plsc.load_gather(gbuf.at[b], [erows[h], offs + j])
                    pbuf[b, j, pl.ds(16 * h, 16)] = v

        def do_block(q, carry):
            b0 = (wid + q * _NW) * 128
            pltpu.sync_copy(xT_hbm.at[:, pl.ds(b0, 128)], idx_v)
            for g in range(FIELDS):
                for h in range(8):
                    v = idx_v[g, pl.ds(16 * h, 16)]
                    ridx_v[g, pl.ds(16 * h, 16)] = lax.shift_right_logical(v, 2)
                    off_v[g, pl.ds(16 * h, 16)] = lax.shift_left(
                        lax.bitwise_and(v, 3), 5
                    )
            for b in range(2):
                pltpu.async_copy(
                    t250_hbm.at[ridx_v.at[b]], gbuf.at[b], gsems[b]
                )

            def fbody(f0, fcarry):
                for b in range(2):
                    f = f0 + b
                    pltpu.make_async_copy(
                        t250_hbm.at[ridx_v.at[f]], gbuf.at[b], gsems[b]
                    ).wait()
                    extract(f, b)
                    pltpu.sync_copy(
                        pbuf.at[b], out_hbm.at[f, :, pl.ds(b0, 128)]
                    )

                    @pl.when(f + 2 < FIELDS)
                    def _():
                        pltpu.async_copy(
                            t250_hbm.at[ridx_v.at[f + 2]], gbuf.at[b], gsems[b]
                        )

                return fcarry

            lax.fori_loop(0, FIELDS // 2, lambda i, c: fbody(i * 2, c), 0)
            return carry

        lax.fori_loop(0, blocks_per_w, do_block, 0)

    return gather_kernel


def kernel(x, table):
    BATCH, FIELDS = x.shape
    V, D = table.shape
    xT = x.astype(jnp.int32).T
    tT = table.T
    ragged = V % 128
    if ragged:
        tail = table[V - ragged:].reshape(ragged // 4, 128)
        t250 = _build_transpose(V, D)(tT, tail)
    else:
        t250 = _build_transpose(V, D)(tT)
    out3 = _build_gather(V, D, BATCH, FIELDS)(t250, xT)
    return out3.transpose(2, 0, 1)


# speedup vs baseline: 4.2965x; 4.2965x over previous
import functools

import jax
import jax.numpy as jnp
from jax import lax
from jax.experimental import pallas as pl
from jax.experimental.pallas import tpu as pltpu
from jax.experimental.pallas import tpu_sc as plsc

_NC = 2
_NS = 16
_NW = _NC * _NS
_L = 16


def _build_transpose(V: int, D: int):
    assert D == 32 and V % 4 == 0
    n_full = V // 128
    main_iters = n_full // _NW
    rem_full = n_full % _NW
    ragged = V % 128
    tail_rows = ragged // 4
    if ragged:
        assert ragged % 4 == 0 and tail_rows % 8 == 0
    mesh = plsc.VectorSubcoreMesh(core_axis_name="c", subcore_axis_name="s")

    def transpose_kernel_body(tT_hbm, tail_hbm, t250_hbm, ibuf, obuf, sem0, sem1):
        wid = lax.axis_index("s") * _NC + lax.axis_index("c")
        sems = (sem0, sem1)
        zi = lax.iota(jnp.int32, 16)
        rvecs = (zi, zi + 16)
        czero = zi * 0

        def shuffle(src, dst, m_rows, s0):
            cvs = [czero + (h // 2) + s0 for h in range(8)]

            @functools.partial(
                plsc.parallel_loop, 0, m_rows, unroll=4
            )
            def _(m):
                m4 = 4 * m
                for h in range(8):
                    v = plsc.load_gather(src, [rvecs[h % 2], cvs[h] + m4])
                    dst[m, pl.ds(16 * h, 16)] = v

        def blk_of(k):
            return wid + k * _NW

        for b in range(2):
            pltpu.async_copy(
                tT_hbm.at[:, pl.ds(blk_of(b) * 128, 128)], ibuf.at[b], sems[b]
            )

        def body(k0, carry):
            s0 = jnp.minimum(k0, 0)
            for b in range(2):
                k = k0 + b
                c = blk_of(k)
                pltpu.make_async_copy(
                    tT_hbm.at[:, pl.ds(c * 128, 128)], ibuf.at[b], sems[b]
                ).wait()
                shuffle(ibuf.at[b], obuf.at[b], 32, s0)
                pltpu.sync_copy(obuf.at[b], t250_hbm.at[pl.ds(c * 32, 32)])

                @pl.when(k + 2 < main_iters)
                def _():
                    pltpu.async_copy(
                        tT_hbm.at[:, pl.ds(blk_of(k + 2) * 128, 128)],
                        ibuf.at[b],
                        sems[b],
                    )

            return carry

        assert main_iters % 2 == 0
        lax.fori_loop(0, main_iters // 2, lambda i, c: body(i * 2, c), 0)

        if rem_full:
            c_tail = (main_iters * _NW) + wid

            @pl.when(wid < rem_full)
            def _():
                pltpu.sync_copy(
                    tT_hbm.at[:, pl.ds(c_tail * 128, 128)], ibuf.at[0]
                )
                shuffle(ibuf.at[0], obuf.at[0], 32, jnp.minimum(wid, 0))
                pltpu.sync_copy(
                    obuf.at[0], t250_hbm.at[pl.ds(c_tail * 32, 32)]
                )

        if ragged:

            @pl.when(wid == rem_full)
            def _():
                pltpu.sync_copy(tail_hbm, obuf.at[0, pl.ds(0, tail_rows)])
                pltpu.sync_copy(
                    obuf.at[0, pl.ds(0, tail_rows)],
                    t250_hbm.at[pl.ds(n_full * 32, tail_rows)],
                )

    if ragged:
        body = transpose_kernel_body
    else:
        def body(tT_hbm, t250_hbm, ibuf, obuf, sem0, sem1):
            dummy = None
            return transpose_kernel_body(
                tT_hbm, dummy, t250_hbm, ibuf, obuf, sem0, sem1
            )

    return functools.partial(
        pl.kernel,
        mesh=mesh,
        compiler_params=pltpu.CompilerParams(needs_layout_passes=False),
        out_type=jax.ShapeDtypeStruct((V // 4, 128), jnp.float32),
        scratch_types=[
            pltpu.VMEM((2, 32, 128), jnp.float32),
            pltpu.VMEM((2, 32, 128), jnp.float32),
            pltpu.SemaphoreType.DMA,
            pltpu.SemaphoreType.DMA,
        ],
    )(body)


def _build_gather(V: int, D: int, BATCH: int, FIELDS: int):
    assert D == 32 and BATCH % (128 * _NW) == 0 and FIELDS % 2 == 0
    blocks_per_w = BATCH // (128 * _NW)
    mesh = plsc.VectorSubcoreMesh(core_axis_name="c", subcore_axis_name="s")

    @functools.partial(
        pl.kernel,
        mesh=mesh,
        compiler_params=pltpu.CompilerParams(needs_layout_passes=False),
        out_type=jax.ShapeDtypeStruct((FIELDS, D, BATCH), jnp.float32),
        scratch_types=[
            pltpu.VMEM((FIELDS, 128), jnp.int32),
            pltpu.VMEM((FIELDS, 128), jnp.int32),
            pltpu.VMEM((FIELDS, 128), jnp.int32),
            pltpu.VMEM((2, 128, 128), jnp.float32),
            pltpu.VMEM((2, 32, 128), jnp.float32),
            pltpu.SemaphoreType.DMA,
            pltpu.SemaphoreType.DMA,
        ],
    )
    def gather_kernel(t250_hbm, xT_hbm, out_hbm, idx_v, ridx_v, off_v,
                      gbuf, pbuf, gsem0, gsem1):
        wid = lax.axis_index("s") * _NC + lax.axis_index("c")
        gsems = (gsem0, gsem1)
        zi = lax.iota(jnp.int32, 16)
        erows = tuple(zi + 16 * h for h in range(8))

        def extract(f, b, s0):
            cvs = [off_v[f, pl.ds(16 * h, 16)] + s0 for h in range(8)]

            @functools.partial(plsc.parallel_loop, 0, 32, unroll=4)
            def _(j):
                for h in range(8):
                    v = plsc.load_gather(gbuf.at[b], [erows[h], cvs[h] + j])
                    pbuf[b, j, pl.ds(16 * h, 16)] = v

        def do_block(q, carry):
            b0 = (wid + q * _NW) * 128
            pltpu.sync_copy(xT_hbm.at[:, pl.ds(b0, 128)], idx_v)
            for g in range(FIELDS):
                for h in range(8):
                    v = idx_v[g, pl.ds(16 * h, 16)]
                    ridx_v[g, pl.ds(16 * h, 16)] = lax.shift_right_logical(v, 2)
                    off_v[g, pl.ds(16 * h, 16)] = lax.shift_left(
                        lax.bitwise_and(v, 3), 5
                    )
            for b in range(2):
                pltpu.async_copy(
                    t250_hbm.at[ridx_v.at[b]], gbuf.at[b], gsems[b]
                )

            def fbody(f0, fcarry):
                s0 = jnp.minimum(f0, 0)
                for b in range(2):
                    f = f0 + b
                    pltpu.make_async_copy(
                        t250_hbm.at[ridx_v.at[f]], gbuf.at[b], gsems[b]
                    ).wait()
                    extract(f, b, s0)
                    pltpu.sync_copy(
                        pbuf.at[b], out_hbm.at[f, :, pl.ds(b0, 128)]
                    )

                    @pl.when(f + 2 < FIELDS)
                    def _():
                        pltpu.async_copy(
                            t250_hbm.at[ridx_v.at[f + 2]], gbuf.at[b], gsems[b]
                        )

                return fcarry

            lax.fori_loop(0, FIELDS // 2, lambda i, c: fbody(i * 2, c), 0)
            return carry

        lax.fori_loop(0, blocks_per_w, do_block, 0)

    return gather_kernel


def kernel(x, table):
    BATCH, FIELDS = x.shape
    V, D = table.shape
    xT = x.astype(jnp.int32).T
    tT = table.T
    ragged = V % 128
    if ragged:
        tail = table[V - ragged:].reshape(ragged // 4, 128)
        t250 = _build_transpose(V, D)(tT, tail)
    else:
        t250 = _build_transpose(V, D)(tT)
    out3 = _build_gather(V, D, BATCH, FIELDS)(t250, xT)
    return out3.transpose(2, 0, 1)
